# baseline (speedup 1.0000x reference)
import functools

import jax
import jax.numpy as jnp
from jax import lax
from jax.experimental import pallas as pl
from jax.experimental.pallas import tpu as pltpu
from jax.experimental.pallas import tpu_sc as plsc

N = 10000
E = 320000
D_IN = 128
D_E = 16
H = 128
G = 64
COMP = 71

NC = 2
NS = 16
NW = NC * NS
EPW = E // NW

GB = 80
SB = 80
RPT = 632
NP = RPT * NS

EPS = 1e-5
F32 = jnp.float32


def _sp(x):
    return jnp.maximum(x, 0.0) + jnp.log1p(jnp.exp(-jnp.abs(x)))


def _scale_shift(ssum, ssq, count, g, beta):
    mean = ssum / count
    var = ssq / count - mean * mean
    scale = g / jnp.sqrt(var + EPS)
    shift = beta - mean * scale
    return jnp.stack([scale, shift]).reshape(2, H).astype(F32)


def _node_enc_body(x_ref, w_ref, gb_ref, wa_ref, wc_ref, h_ref, ha_ref, hc_ref):
    y = jnp.dot(x_ref[...], w_ref[...], preferred_element_type=F32)
    m = jnp.mean(y, axis=0, keepdims=True)
    v = jnp.mean((y - m) * (y - m), axis=0, keepdims=True)
    h = _sp(gb_ref[0:1, :] * (y - m) / jnp.sqrt(v + EPS) + gb_ref[1:2, :])
    h_ref[...] = h
    ha_ref[...] = jnp.dot(h, wa_ref[...], preferred_element_type=F32)
    hc_ref[...] = jnp.dot(h, wc_ref[...], preferred_element_type=F32)


def _node_enc(x, w, gb, wa, wc):
    return pl.pallas_call(
        _node_enc_body,
        out_shape=[jax.ShapeDtypeStruct((N, H), F32)] * 3,
    )(x, w, gb, wa, wc)


_GRAM_B = 4000


def _edge_stats_body(a_ref, w_ref, st_ref):
    i = pl.program_id(0)
    y = jnp.dot(a_ref[...], w_ref[...], preferred_element_type=F32)

    @pl.when(i == 0)
    def _():
        st_ref[...] = jnp.zeros_like(st_ref)

    st_ref[0:1, :] = st_ref[0:1, :] + jnp.sum(y, axis=0, keepdims=True)
    st_ref[1:2, :] = st_ref[1:2, :] + jnp.sum(y * y, axis=0, keepdims=True)


def _edge_stats(edge_attr, w):
    return pl.pallas_call(
        _edge_stats_body,
        grid=(E // _GRAM_B,),
        in_specs=[pl.BlockSpec((_GRAM_B, D_E), lambda i: (i, 0)),
                  pl.BlockSpec((D_E, H), lambda i: (0, 0))],
        out_specs=pl.BlockSpec((8, H), lambda i: (0, 0)),
        out_shape=jax.ShapeDtypeStruct((8, H), F32),
    )(edge_attr, w)


_EAB = 2000


def _edge_apply_body(a_ref, w_ref, ss_ref, wb0_ref, wb1_ref, wb2_ref, wb3_ref,
                     o0_ref, o1_ref, o2_ref, o3_ref):
    y = jnp.dot(a_ref[...], w_ref[...], preferred_element_type=F32)
    e = _sp(y * ss_ref[0:1, :] + ss_ref[1:2, :])
    o0_ref[...] = jnp.dot(e, wb0_ref[...], preferred_element_type=F32)
    o1_ref[...] = jnp.dot(e, wb1_ref[...], preferred_element_type=F32)
    o2_ref[...] = jnp.dot(e, wb2_ref[...], preferred_element_type=F32)
    o3_ref[...] = jnp.dot(e, wb3_ref[...], preferred_element_type=F32)


def _edge_apply(edge_attr, w, ss, wbs):
    blk = pl.BlockSpec((_EAB, H), lambda i: (i, 0))
    full = lambda shape: pl.BlockSpec(shape, lambda i: (0, 0))
    return pl.pallas_call(
        _edge_apply_body,
        grid=(E // _EAB,),
        in_specs=[pl.BlockSpec((_EAB, D_E), lambda i: (i, 0)),
                  full((D_E, H)), full((2, H)),
                  full((H, H)), full((H, H)), full((H, H)), full((H, H))],
        out_specs=[blk, blk, blk, blk],
        out_shape=[jax.ShapeDtypeStruct((E, H), F32)] * 4,
    )(edge_attr, w, ss, *wbs)


_P2B = 2000


def _p2s_body(y1_ref, ss_ref, gram_ref, cs_ref):
    i = pl.program_id(0)
    m1 = _sp(y1_ref[...] * ss_ref[0:1, :] + ss_ref[1:2, :])
    g = lax.dot_general(m1, m1, (((0,), (0,)), ((), ())),
                        preferred_element_type=F32,
                        precision=lax.Precision.HIGHEST)

    @pl.when(i == 0)
    def _():
        gram_ref[...] = jnp.zeros_like(gram_ref)
        cs_ref[...] = jnp.zeros_like(cs_ref)

    gram_ref[...] = gram_ref[...] + g
    cs_ref[0:1, :] = cs_ref[0:1, :] + jnp.sum(m1, axis=0, keepdims=True)


def _p2s(y1, ss):
    blk = pl.BlockSpec((_P2B, H), lambda i: (i, 0))
    full = lambda shape: pl.BlockSpec(shape, lambda i: (0, 0))
    return pl.pallas_call(
        _p2s_body,
        grid=(E // _P2B,),
        in_specs=[blk, full((2, H))],
        out_specs=[full((H, H)), full((8, H))],
        out_shape=[jax.ShapeDtypeStruct((H, H), F32),
                   jax.ShapeDtypeStruct((8, H), F32)],
    )(y1, ss)


def _p2a_body(y1_ref, ss1_ref, ss2_ref, w_ref, m_ref):
    m1 = _sp(y1_ref[...] * ss1_ref[0:1, :] + ss1_ref[1:2, :])
    y2 = jnp.dot(m1, w_ref[...], preferred_element_type=F32)
    m_ref[...] = _sp(y2 * ss2_ref[0:1, :] + ss2_ref[1:2, :])


def _p2a(y1, ss1, ss2, w):
    blk = pl.BlockSpec((_P2B, H), lambda i: (i, 0))
    full = lambda shape: pl.BlockSpec(shape, lambda i: (0, 0))
    return pl.pallas_call(
        _p2a_body,
        grid=(E // _P2B,),
        in_specs=[blk, full((2, H)), full((2, H)), full((H, H))],
        out_specs=blk,
        out_shape=jax.ShapeDtypeStruct((E, H), F32),
    )(y1, ss1, ss2, w)


def _p4_body(h_ref, ag_ref, w_ref, gb_ref, wa_ref, wc_ref,
             h2_ref, ha_ref, hc_ref):
    hin = h_ref[...] + ag_ref[0:N, :] + ag_ref[NP:NP + N, :]
    z = jnp.dot(hin, w_ref[...], preferred_element_type=F32)
    m = jnp.mean(z, axis=0, keepdims=True)
    v = jnp.mean((z - m) * (z - m), axis=0, keepdims=True)
    h2 = _sp(gb_ref[0:1, :] * (z - m) / jnp.sqrt(v + EPS) + gb_ref[1:2, :])
    h2_ref[...] = h2
    ha_ref[...] = jnp.dot(h2, wa_ref[...], preferred_element_type=F32)
    hc_ref[...] = jnp.dot(h2, wc_ref[...], preferred_element_type=F32)


def _p4(h, aggp, w, gb, wa, wc):
    return pl.pallas_call(
        _p4_body,
        out_shape=[jax.ShapeDtypeStruct((N, H), F32)] * 3,
    )(h, aggp, w, gb, wa, wc)


def _final_body(h_ref, ag_ref, w3_ref, gb3_ref, batch_ref, comp_ref,
                cmpw_ref, cmpgb_ref, fc1w_ref, fc1gb_ref, fc2w_ref, fc2gb_ref,
                outw_ref, outb_ref, o_ref):
    hin = h_ref[...] + ag_ref[0:N, :] + ag_ref[NP:NP + N, :]
    z = jnp.dot(hin, w3_ref[...], preferred_element_type=F32)
    m = jnp.mean(z, axis=0, keepdims=True)
    v = jnp.mean((z - m) * (z - m), axis=0, keepdims=True)
    h4 = _sp(gb3_ref[0:1, :] * (z - m) / jnp.sqrt(v + EPS) + gb3_ref[1:2, :])

    seg = jax.lax.broadcasted_iota(jnp.int32, (N, G), 1)
    p = (batch_ref[...] == seg).astype(F32)
    sums = lax.dot_general(p, h4, (((0,), (0,)), ((), ())),
                           preferred_element_type=F32,
                           precision=lax.Precision.HIGHEST)
    cnt = jnp.sum(p, axis=0)[:, None]
    gmean = sums / jnp.maximum(cnt, 1.0)

    def bn_sp(y, gb):
        mm = jnp.mean(y, axis=0, keepdims=True)
        vv = jnp.mean((y - mm) * (y - mm), axis=0, keepdims=True)
        return _sp(gb[0:1, :] * (y - mm) / jnp.sqrt(vv + EPS) + gb[1:2, :])

    cf = bn_sp(jnp.dot(comp_ref[...], cmpw_ref[...],
                       preferred_element_type=F32), cmpgb_ref[...])
    g1 = jnp.concatenate([gmean, cf], axis=1)
    g2 = bn_sp(jnp.dot(g1, fc1w_ref[...], preferred_element_type=F32),
               fc1gb_ref[...])
    g3 = bn_sp(jnp.dot(g2, fc2w_ref[...], preferred_element_type=F32),
               fc2gb_ref[...])
    o_ref[...] = (jnp.dot(g3, outw_ref[...], preferred_element_type=F32)
                  + outb_ref[0:1, :])


def _final(h, aggp, w3, gb3, batch2d, comp, cmpw, cmpgb, fc1w, fc1gb,
           fc2w, fc2gb, outw, outb):
    return pl.pallas_call(
        _final_body,
        out_shape=jax.ShapeDtypeStruct((G, 1), F32),
    )(h, aggp, w3, gb3, batch2d, comp, cmpw, cmpgb, fc1w, fc1gb,
      fc2w, fc2gb, outw, outb)


@functools.cache
def _mesh():
    return plsc.VectorSubcoreMesh(core_axis_name="c", subcore_axis_name="s",
                                  num_cores=NC, num_subcores=NS)


_NCH = EPW // GB


def _gather_body(ha_hbm, hc_hbm, eb_hbm, row_hbm, col_hbm,
                 y1_hbm, stats_hbm,
                 idxr, idxc, bufa, bufc, bufe, bufy, sbuf, sems):
    cid = lax.axis_index("c")
    sid = lax.axis_index("s")
    wid = sid * NC + cid
    base0 = wid * EPW

    pltpu.sync_copy(row_hbm.at[wid], idxr)
    pltpu.sync_copy(col_hbm.at[wid], idxc)

    def issue(c, b):
        base = base0 + c * GB
        isl = pl.ds(c * GB, GB)
        pltpu.async_copy(ha_hbm.at[idxr.at[isl]], bufa.at[b], sems.at[b, 0])
        pltpu.async_copy(hc_hbm.at[idxc.at[isl]], bufc.at[b], sems.at[b, 1])
        pltpu.async_copy(eb_hbm.at[pl.ds(base, GB)], bufe.at[b], sems.at[b, 2])

    def wait_in(b):
        pltpu.make_async_copy(eb_hbm.at[pl.ds(0, GB)], bufa.at[b],
                              sems.at[b, 0]).wait()
        pltpu.make_async_copy(eb_hbm.at[pl.ds(0, GB)], bufc.at[b],
                              sems.at[b, 1]).wait()
        pltpu.make_async_copy(eb_hbm.at[pl.ds(0, GB)], bufe.at[b],
                              sems.at[b, 2]).wait()

    def wait_wb(b):
        pltpu.make_async_copy(eb_hbm.at[pl.ds(0, GB)], bufy.at[b],
                              sems.at[b, 3]).wait()

    def compute(c, b, accs):
        def rowloop(r, acc):
            acc_s, acc_q = acc
            ns, nq = [], []
            for k in range(H // 16):
                sl = pl.ds(16 * k, 16)
                y = bufa[b, r, sl] + bufc[b, r, sl] + bufe[b, r, sl]
                bufy[b, r, sl] = y
                ns.append(acc_s[k] + y)
                nq.append(acc_q[k] + y * y)
            return (tuple(ns), tuple(nq))

        accs = lax.fori_loop(0, GB, rowloop, accs)
        pltpu.async_copy(bufy.at[b], y1_hbm.at[pl.ds(base0 + c * GB, GB)],
                         sems.at[b, 3])
        return accs

    z = jnp.zeros((16,), F32)
    accs = (tuple(z for _ in range(H // 16)), tuple(z for _ in range(H // 16)))

    issue(0, 0)
    issue(1, 1)

    def outer(k2, accs):
        for j in range(2):
            c = 2 * k2 + j
            wait_in(j)

            @pl.when(k2 > 0)
            def _():
                wait_wb(j)

            accs = compute(c, j, accs)

            @pl.when(c + 2 < _NCH)
            def _():
                issue(c + 2, j)
        return accs

    accs = lax.fori_loop(0, (_NCH - 1) // 2, outer, accs)
    wait_in(0)
    wait_wb(0)
    accs = compute(_NCH - 1, 0, accs)
    wait_wb(1)
    wait_wb(0)

    acc_s, acc_q = accs
    for k in range(H // 16):
        sl = pl.ds(16 * k, 16)
        sbuf[0, sl] = acc_s[k]
        sbuf[1, sl] = acc_q[k]
    pltpu.sync_copy(sbuf, stats_hbm.at[pl.ds(wid * 8, 8)])


@functools.cache
def _gather_call():
    return pl.kernel(
        _gather_body,
        out_type=[jax.ShapeDtypeStruct((E, H), F32),
                  jax.ShapeDtypeStruct((NW * 8, H), F32)],
        mesh=_mesh(),
        scratch_types=[
            pltpu.VMEM((EPW,), jnp.int32),
            pltpu.VMEM((EPW,), jnp.int32),
            pltpu.VMEM((2, GB, H), F32),
            pltpu.VMEM((2, GB, H), F32),
            pltpu.VMEM((2, GB, H), F32),
            pltpu.VMEM((2, GB, H), F32),
            pltpu.VMEM((8, H), F32),
            pltpu.SemaphoreType.DMA((2, 4)),
        ],
    )


_NCHS = EPW // SB


def _scatter_body(m_hbm, col_hbm, z_hbm, aggp_hbm, shared, idxall, mbuf, sems):
    cid = lax.axis_index("c")
    sid = lax.axis_index("s")
    wid = sid * NC + cid
    base0 = wid * EPW
    pltpu.sync_copy(col_hbm.at[wid], idxall)
    pltpu.sync_copy(z_hbm.at[pl.ds(sid * RPT, RPT)],
                    shared.at[pl.ds(sid * RPT, RPT)])
    plsc.subcore_barrier()

    def issue(c, b):
        pltpu.async_copy(m_hbm.at[pl.ds(base0 + c * SB, SB)], mbuf.at[b],
                         sems.at[b])

    def wait_in(b):
        pltpu.make_async_copy(m_hbm.at[pl.ds(0, SB)], mbuf.at[b],
                              sems.at[b]).wait()

    issue(0, 0)
    issue(1, 1)

    def outer(k2, carry):
        for j in range(2):
            c = 2 * k2 + j
            wait_in(j)
            pltpu.sync_copy(mbuf.at[j], shared.at[idxall.at[c]], add=True)

            @pl.when(c + 2 < _NCHS)
            def _():
                issue(c + 2, j)
        return carry

    lax.fori_loop(0, (_NCHS - 1) // 2, outer, 0)
    wait_in(0)
    pltpu.sync_copy(mbuf.at[0], shared.at[idxall.at[_NCHS - 1]], add=True)
    plsc.subcore_barrier()
    pltpu.sync_copy(shared.at[pl.ds(sid * RPT, RPT)],
                    aggp_hbm.at[pl.ds(cid * NP + sid * RPT, RPT)])


@functools.cache
def _scatter_call():
    return pl.kernel(
        _scatter_body,
        out_type=jax.ShapeDtypeStruct((NC * NP, H), F32),
        mesh=_mesh(),
        scratch_types=[
            pltpu.VMEM_SHARED((NP, H), F32),
            pltpu.VMEM((_NCHS, SB), jnp.int32),
            pltpu.VMEM((2, SB, H), F32),
            pltpu.SemaphoreType.DMA((2,)),
        ],
    )


def kernel(x, edge_attr, comp_features, params, edge_index, batch):
    p = params
    x = x.astype(F32)
    edge_attr = edge_attr.astype(F32)
    comp_features = comp_features.astype(F32)
    row = edge_index[0].astype(jnp.int32)
    col = edge_index[1].astype(jnp.int32)
    row2 = row.reshape(NW, EPW)
    col2 = col.reshape(NW, EPW)
    col3 = col.reshape(NW, _NCHS, SB)
    batch2d = batch.astype(jnp.int32).reshape(N, 1)
    zeros_n = jnp.zeros((NP, H), F32)

    convs = p['convs']
    was = [c['e1_W'][0:H].astype(F32) for c in convs]
    wbs = [c['e1_W'][H:2 * H].astype(F32) for c in convs]
    wcs = [c['e1_W'][2 * H:3 * H].astype(F32) for c in convs]

    def gb(gname, bname, src):
        return jnp.stack([src[gname], src[bname]]).reshape(2, H).astype(F32)

    ne_gb = jnp.stack([p['ne_g'], p['ne_beta']]).reshape(2, H).astype(F32)
    h, ha, hc = _node_enc(x, p['ne_W'].astype(F32), ne_gb, was[0], wcs[0])

    w_ee = p['ee_W'].astype(F32)
    est = _edge_stats(edge_attr, w_ee)
    mean_y = est[0] / E
    var_y = est[1] / E - mean_y * mean_y
    scale = p['ee_g'] / jnp.sqrt(var_y + EPS)
    shift = p['ee_beta'] - mean_y * scale
    ee_ss = jnp.stack([scale, shift]).reshape(2, H).astype(F32)
    ebs = _edge_apply(edge_attr, w_ee, ee_ss, wbs)

    for l in range(4):
        c = convs[l]
        y1, stats = _gather_call()(ha, hc, ebs[l], row2, col2)
        st = stats.reshape(NW, 8, H)
        ss1 = _scale_shift(jnp.sum(st[:, 0], axis=0), jnp.sum(st[:, 1], axis=0),
                           float(E), c['bn1_g'], c['bn1_b'])
        w2 = c['e2_W'].astype(F32)
        gram2, cs2 = _p2s(y1, ss1)
        mean2 = jnp.dot(cs2[0:1] / E, w2,
                        precision=lax.Precision.HIGHEST)[0]
        ey2 = jnp.sum(jnp.dot(gram2, w2,
                              precision=lax.Precision.HIGHEST) * w2,
                      axis=0) / E
        var2 = ey2 - mean2 * mean2
        scale2 = c['bn2_g'] / jnp.sqrt(var2 + EPS)
        shift2 = c['bn2_b'] - mean2 * scale2
        ss2 = jnp.stack([scale2, shift2]).reshape(2, H).astype(F32)
        msg = _p2a(y1, ss1, ss2, w2)
        aggp = _scatter_call()(msg, col3, zeros_n)
        gb3 = gb('bn3_g', 'bn3_b', c)
        if l < 3:
            h, ha, hc = _p4(h, aggp, c['n_W'].astype(F32), gb3,
                            was[l + 1], wcs[l + 1])
        else:
            cmp_gb = gb('cmp_g', 'cmp_beta', p)
            fc1_gb = gb('fc1_g', 'fc1_beta', p)
            fc2_gb = gb('fc2_g', 'fc2_beta', p)
            out = _final(h, aggp, c['n_W'].astype(F32), gb3, batch2d,
                         comp_features, p['cmp_W'].astype(F32), cmp_gb,
                         p['fc1_W'].astype(F32), fc1_gb,
                         p['fc2_W'].astype(F32), fc2_gb,
                         p['out_W'].astype(F32),
                         p['out_b'].reshape(1, 1).astype(F32))
    return out


# speedup vs baseline: 1.1128x; 1.1128x over previous
import functools

import jax
import jax.numpy as jnp
from jax import lax
from jax.experimental import pallas as pl
from jax.experimental.pallas import tpu as pltpu
from jax.experimental.pallas import tpu_sc as plsc

N = 10000
E = 320000
D_IN = 128
D_E = 16
H = 128
G = 64
COMP = 71

NC = 2
NS = 16
NW = NC * NS
EPW = E // NW

GB = 80
SB = 80
RPT = 632
NP = RPT * NS

EPS = 1e-5
F32 = jnp.float32


def _sp(x):
    return jnp.maximum(x, 0.0) + jnp.log1p(jnp.exp(-jnp.abs(x)))


def _scale_shift(ssum, ssq, count, g, beta):
    mean = ssum / count
    var = ssq / count - mean * mean
    scale = g / jnp.sqrt(var + EPS)
    shift = beta - mean * scale
    return jnp.stack([scale, shift]).reshape(2, H).astype(F32)


def _node_enc_body(x_ref, w_ref, gb_ref, wa_ref, wc_ref, h_ref, ha_ref, hc_ref):
    y = jnp.dot(x_ref[...], w_ref[...], preferred_element_type=F32)
    m = jnp.mean(y, axis=0, keepdims=True)
    v = jnp.mean((y - m) * (y - m), axis=0, keepdims=True)
    h = _sp(gb_ref[0:1, :] * (y - m) / jnp.sqrt(v + EPS) + gb_ref[1:2, :])
    h_ref[...] = h
    ha_ref[...] = jnp.dot(h, wa_ref[...], preferred_element_type=F32)
    hc_ref[...] = jnp.dot(h, wc_ref[...], preferred_element_type=F32)


def _node_enc(x, w, gb, wa, wc):
    return pl.pallas_call(
        _node_enc_body,
        out_shape=[jax.ShapeDtypeStruct((N, H), F32)] * 3,
    )(x, w, gb, wa, wc)


_GRAM_B = 4000


def _edge_stats_body(a_ref, w_ref, st_ref):
    i = pl.program_id(0)
    y = jnp.dot(a_ref[...], w_ref[...], preferred_element_type=F32)

    @pl.when(i == 0)
    def _():
        st_ref[...] = jnp.zeros_like(st_ref)

    st_ref[0:1, :] = st_ref[0:1, :] + jnp.sum(y, axis=0, keepdims=True)
    st_ref[1:2, :] = st_ref[1:2, :] + jnp.sum(y * y, axis=0, keepdims=True)


def _edge_stats(edge_attr, w):
    return pl.pallas_call(
        _edge_stats_body,
        grid=(E // _GRAM_B,),
        in_specs=[pl.BlockSpec((_GRAM_B, D_E), lambda i: (i, 0)),
                  pl.BlockSpec((D_E, H), lambda i: (0, 0))],
        out_specs=pl.BlockSpec((8, H), lambda i: (0, 0)),
        out_shape=jax.ShapeDtypeStruct((8, H), F32),
    )(edge_attr, w)


_EAB = 2000


def _edge_apply_body(a_ref, w_ref, ss_ref, wb_ref, o_ref):
    y = jnp.dot(a_ref[...], w_ref[...], preferred_element_type=F32)
    e = _sp(y * ss_ref[0:1, :] + ss_ref[1:2, :])
    o_ref[...] = jnp.dot(e, wb_ref[...], preferred_element_type=F32)


def _edge_apply(edge_attr, w, ss, wb):
    full = lambda shape: pl.BlockSpec(shape, lambda i: (0, 0))
    return pl.pallas_call(
        _edge_apply_body,
        grid=(E // _EAB,),
        in_specs=[pl.BlockSpec((_EAB, D_E), lambda i: (i, 0)),
                  full((D_E, H)), full((2, H)), full((H, H))],
        out_specs=pl.BlockSpec((_EAB, H), lambda i: (i, 0)),
        out_shape=jax.ShapeDtypeStruct((E, H), F32),
    )(edge_attr, w, ss, wb)


_P2B = 2000


def _p2_body(y1_ref, ss_ref, w_ref, y2_ref, st_ref):
    i = pl.program_id(0)
    m1 = _sp(y1_ref[...] * ss_ref[0:1, :] + ss_ref[1:2, :])
    y2 = jnp.dot(m1, w_ref[...], preferred_element_type=F32)
    y2_ref[...] = y2

    @pl.when(i == 0)
    def _():
        st_ref[...] = jnp.zeros_like(st_ref)

    st_ref[0:1, :] = st_ref[0:1, :] + jnp.sum(y2, axis=0, keepdims=True)
    st_ref[1:2, :] = st_ref[1:2, :] + jnp.sum(y2 * y2, axis=0, keepdims=True)


def _p2(y1, ss, w):
    blk = pl.BlockSpec((_P2B, H), lambda i: (i, 0))
    full = lambda shape: pl.BlockSpec(shape, lambda i: (0, 0))
    return pl.pallas_call(
        _p2_body,
        grid=(E // _P2B,),
        in_specs=[blk, full((2, H)), full((H, H))],
        out_specs=[blk, full((8, H))],
        out_shape=[jax.ShapeDtypeStruct((E, H), F32),
                   jax.ShapeDtypeStruct((8, H), F32)],
    )(y1, ss, w)


def _p3_body(y2_ref, ss_ref, m_ref):
    m_ref[...] = _sp(y2_ref[...] * ss_ref[0:1, :] + ss_ref[1:2, :])


def _p3(y2, ss):
    blk = pl.BlockSpec((_P2B, H), lambda i: (i, 0))
    return pl.pallas_call(
        _p3_body,
        grid=(E // _P2B,),
        in_specs=[blk, pl.BlockSpec((2, H), lambda i: (0, 0))],
        out_specs=blk,
        out_shape=jax.ShapeDtypeStruct((E, H), F32),
    )(y2, ss)


def _p4_body(h_ref, ag_ref, w_ref, gb_ref, wa_ref, wc_ref,
             h2_ref, ha_ref, hc_ref):
    hin = h_ref[...] + ag_ref[0:N, :] + ag_ref[NP:NP + N, :]
    z = jnp.dot(hin, w_ref[...], preferred_element_type=F32)
    m = jnp.mean(z, axis=0, keepdims=True)
    v = jnp.mean((z - m) * (z - m), axis=0, keepdims=True)
    h2 = _sp(gb_ref[0:1, :] * (z - m) / jnp.sqrt(v + EPS) + gb_ref[1:2, :])
    h2_ref[...] = h2
    ha_ref[...] = jnp.dot(h2, wa_ref[...], preferred_element_type=F32)
    hc_ref[...] = jnp.dot(h2, wc_ref[...], preferred_element_type=F32)


def _p4(h, aggp, w, gb, wa, wc):
    return pl.pallas_call(
        _p4_body,
        out_shape=[jax.ShapeDtypeStruct((N, H), F32)] * 3,
    )(h, aggp, w, gb, wa, wc)


def _final_body(h_ref, ag_ref, w3_ref, gb3_ref, batch_ref, comp_ref,
                cmpw_ref, cmpgb_ref, fc1w_ref, fc1gb_ref, fc2w_ref, fc2gb_ref,
                outw_ref, outb_ref, o_ref):
    hin = h_ref[...] + ag_ref[0:N, :] + ag_ref[NP:NP + N, :]
    z = jnp.dot(hin, w3_ref[...], preferred_element_type=F32)
    m = jnp.mean(z, axis=0, keepdims=True)
    v = jnp.mean((z - m) * (z - m), axis=0, keepdims=True)
    h4 = _sp(gb3_ref[0:1, :] * (z - m) / jnp.sqrt(v + EPS) + gb3_ref[1:2, :])

    seg = jax.lax.broadcasted_iota(jnp.int32, (N, G), 1)
    p = (batch_ref[...] == seg).astype(F32)
    sums = lax.dot_general(p, h4, (((0,), (0,)), ((), ())),
                           preferred_element_type=F32,
                           precision=lax.Precision.HIGHEST)
    cnt = jnp.sum(p, axis=0)[:, None]
    gmean = sums / jnp.maximum(cnt, 1.0)

    def bn_sp(y, gb):
        mm = jnp.mean(y, axis=0, keepdims=True)
        vv = jnp.mean((y - mm) * (y - mm), axis=0, keepdims=True)
        return _sp(gb[0:1, :] * (y - mm) / jnp.sqrt(vv + EPS) + gb[1:2, :])

    cf = bn_sp(jnp.dot(comp_ref[...], cmpw_ref[...],
                       preferred_element_type=F32), cmpgb_ref[...])
    g1 = jnp.concatenate([gmean, cf], axis=1)
    g2 = bn_sp(jnp.dot(g1, fc1w_ref[...], preferred_element_type=F32),
               fc1gb_ref[...])
    g3 = bn_sp(jnp.dot(g2, fc2w_ref[...], preferred_element_type=F32),
               fc2gb_ref[...])
    o_ref[...] = (jnp.dot(g3, outw_ref[...], preferred_element_type=F32)
                  + outb_ref[0:1, :])


def _final(h, aggp, w3, gb3, batch2d, comp, cmpw, cmpgb, fc1w, fc1gb,
           fc2w, fc2gb, outw, outb):
    return pl.pallas_call(
        _final_body,
        out_shape=jax.ShapeDtypeStruct((G, 1), F32),
    )(h, aggp, w3, gb3, batch2d, comp, cmpw, cmpgb, fc1w, fc1gb,
      fc2w, fc2gb, outw, outb)


@functools.cache
def _mesh():
    return plsc.VectorSubcoreMesh(core_axis_name="c", subcore_axis_name="s",
                                  num_cores=NC, num_subcores=NS)


_NCH = EPW // GB


def _gather_body(ha_hbm, hc_hbm, eb_hbm, row_hbm, col_hbm,
                 y1_hbm, stats_hbm,
                 idxr, idxc, bufa, bufc, bufe, bufy, sbuf, sems):
    cid = lax.axis_index("c")
    sid = lax.axis_index("s")
    wid = sid * NC + cid
    base0 = wid * EPW

    pltpu.sync_copy(row_hbm.at[wid], idxr)
    pltpu.sync_copy(col_hbm.at[wid], idxc)

    def issue(c, b):
        base = base0 + c * GB
        isl = pl.ds(c * GB, GB)
        pltpu.async_copy(ha_hbm.at[idxr.at[isl]], bufa.at[b], sems.at[b, 0])
        pltpu.async_copy(hc_hbm.at[idxc.at[isl]], bufc.at[b], sems.at[b, 1])
        pltpu.async_copy(eb_hbm.at[pl.ds(base, GB)], bufe.at[b], sems.at[b, 2])

    def wait_in(b):
        pltpu.make_async_copy(eb_hbm.at[pl.ds(0, GB)], bufa.at[b],
                              sems.at[b, 0]).wait()
        pltpu.make_async_copy(eb_hbm.at[pl.ds(0, GB)], bufc.at[b],
                              sems.at[b, 1]).wait()
        pltpu.make_async_copy(eb_hbm.at[pl.ds(0, GB)], bufe.at[b],
                              sems.at[b, 2]).wait()

    def wait_wb(b):
        pltpu.make_async_copy(eb_hbm.at[pl.ds(0, GB)], bufy.at[b],
                              sems.at[b, 3]).wait()

    def compute(c, b, accs):
        def rowloop(r, acc):
            acc_s, acc_q = acc
            ns, nq = [], []
            for k in range(H // 16):
                sl = pl.ds(16 * k, 16)
                y = bufa[b, r, sl] + bufc[b, r, sl] + bufe[b, r, sl]
                bufy[b, r, sl] = y
                ns.append(acc_s[k] + y)
                nq.append(acc_q[k] + y * y)
            return (tuple(ns), tuple(nq))

        accs = lax.fori_loop(0, GB, rowloop, accs)
        pltpu.async_copy(bufy.at[b], y1_hbm.at[pl.ds(base0 + c * GB, GB)],
                         sems.at[b, 3])
        return accs

    z = jnp.zeros((16,), F32)
    accs = (tuple(z for _ in range(H // 16)), tuple(z for _ in range(H // 16)))

    issue(0, 0)
    issue(1, 1)

    def outer(k2, accs):
        for j in range(2):
            c = 2 * k2 + j
            wait_in(j)

            @pl.when(k2 > 0)
            def _():
                wait_wb(j)

            accs = compute(c, j, accs)

            @pl.when(c + 2 < _NCH)
            def _():
                issue(c + 2, j)
        return accs

    accs = lax.fori_loop(0, (_NCH - 1) // 2, outer, accs)
    wait_in(0)
    wait_wb(0)
    accs = compute(_NCH - 1, 0, accs)
    wait_wb(1)
    wait_wb(0)

    acc_s, acc_q = accs
    for k in range(H // 16):
        sl = pl.ds(16 * k, 16)
        sbuf[0, sl] = acc_s[k]
        sbuf[1, sl] = acc_q[k]
    pltpu.sync_copy(sbuf, stats_hbm.at[pl.ds(wid * 8, 8)])


@functools.cache
def _gather_call():
    return pl.kernel(
        _gather_body,
        out_type=[jax.ShapeDtypeStruct((E, H), F32),
                  jax.ShapeDtypeStruct((NW * 8, H), F32)],
        mesh=_mesh(),
        scratch_types=[
            pltpu.VMEM((EPW,), jnp.int32),
            pltpu.VMEM((EPW,), jnp.int32),
            pltpu.VMEM((2, GB, H), F32),
            pltpu.VMEM((2, GB, H), F32),
            pltpu.VMEM((2, GB, H), F32),
            pltpu.VMEM((2, GB, H), F32),
            pltpu.VMEM((8, H), F32),
            pltpu.SemaphoreType.DMA((2, 4)),
        ],
    )


_NCHS = EPW // SB


def _scatter_body(m_hbm, col_hbm, z_hbm, aggp_hbm, shared, idxall, mbuf, sems):
    cid = lax.axis_index("c")
    sid = lax.axis_index("s")
    wid = sid * NC + cid
    base0 = wid * EPW
    pltpu.sync_copy(col_hbm.at[wid], idxall)
    pltpu.sync_copy(z_hbm.at[pl.ds(sid * RPT, RPT)],
                    shared.at[pl.ds(sid * RPT, RPT)])
    plsc.subcore_barrier()

    def issue(c, b):
        pltpu.async_copy(m_hbm.at[pl.ds(base0 + c * SB, SB)], mbuf.at[b],
                         sems.at[b])

    def wait_in(b):
        pltpu.make_async_copy(m_hbm.at[pl.ds(0, SB)], mbuf.at[b],
                              sems.at[b]).wait()

    issue(0, 0)
    issue(1, 1)

    def outer(k2, carry):
        for j in range(2):
            c = 2 * k2 + j
            wait_in(j)
            pltpu.sync_copy(mbuf.at[j], shared.at[idxall.at[c]], add=True)

            @pl.when(c + 2 < _NCHS)
            def _():
                issue(c + 2, j)
        return carry

    lax.fori_loop(0, (_NCHS - 1) // 2, outer, 0)
    wait_in(0)
    pltpu.sync_copy(mbuf.at[0], shared.at[idxall.at[_NCHS - 1]], add=True)
    plsc.subcore_barrier()
    pltpu.sync_copy(shared.at[pl.ds(sid * RPT, RPT)],
                    aggp_hbm.at[pl.ds(cid * NP + sid * RPT, RPT)])


@functools.cache
def _scatter_call():
    return pl.kernel(
        _scatter_body,
        out_type=jax.ShapeDtypeStruct((NC * NP, H), F32),
        mesh=_mesh(),
        scratch_types=[
            pltpu.VMEM_SHARED((NP, H), F32),
            pltpu.VMEM((_NCHS, SB), jnp.int32),
            pltpu.VMEM((2, SB, H), F32),
            pltpu.SemaphoreType.DMA((2,)),
        ],
    )


def kernel(x, edge_attr, comp_features, params, edge_index, batch):
    p = params
    x = x.astype(F32)
    edge_attr = edge_attr.astype(F32)
    comp_features = comp_features.astype(F32)
    row = edge_index[0].astype(jnp.int32)
    col = edge_index[1].astype(jnp.int32)
    row2 = row.reshape(NW, EPW)
    col2 = col.reshape(NW, EPW)
    col3 = col.reshape(NW, _NCHS, SB)
    batch2d = batch.astype(jnp.int32).reshape(N, 1)
    zeros_n = jnp.zeros((NP, H), F32)

    convs = p['convs']
    was = [c['e1_W'][0:H].astype(F32) for c in convs]
    wbs = [c['e1_W'][H:2 * H].astype(F32) for c in convs]
    wcs = [c['e1_W'][2 * H:3 * H].astype(F32) for c in convs]

    def gb(gname, bname, src):
        return jnp.stack([src[gname], src[bname]]).reshape(2, H).astype(F32)

    ne_gb = jnp.stack([p['ne_g'], p['ne_beta']]).reshape(2, H).astype(F32)
    h, ha, hc = _node_enc(x, p['ne_W'].astype(F32), ne_gb, was[0], wcs[0])

    w_ee = p['ee_W'].astype(F32)
    est = _edge_stats(edge_attr, w_ee)
    mean_y = est[0] / E
    var_y = est[1] / E - mean_y * mean_y
    scale = p['ee_g'] / jnp.sqrt(var_y + EPS)
    shift = p['ee_beta'] - mean_y * scale
    ee_ss = jnp.stack([scale, shift]).reshape(2, H).astype(F32)
    ebs = [_edge_apply(edge_attr, w_ee, ee_ss, wb) for wb in wbs]

    for l in range(4):
        c = convs[l]
        y1, stats = _gather_call()(ha, hc, ebs[l], row2, col2)
        st = stats.reshape(NW, 8, H)
        ss1 = _scale_shift(jnp.sum(st[:, 0], axis=0), jnp.sum(st[:, 1], axis=0),
                           float(E), c['bn1_g'], c['bn1_b'])
        y2, st2 = _p2(y1, ss1, c['e2_W'].astype(F32))
        ss2 = _scale_shift(st2[0], st2[1], float(E), c['bn2_g'], c['bn2_b'])
        msg = _p3(y2, ss2)
        aggp = _scatter_call()(msg, col3, zeros_n)
        gb3 = gb('bn3_g', 'bn3_b', c)
        if l < 3:
            h, ha, hc = _p4(h, aggp, c['n_W'].astype(F32), gb3,
                            was[l + 1], wcs[l + 1])
        else:
            cmp_gb = gb('cmp_g', 'cmp_beta', p)
            fc1_gb = gb('fc1_g', 'fc1_beta', p)
            fc2_gb = gb('fc2_g', 'fc2_beta', p)
            out = _final(h, aggp, c['n_W'].astype(F32), gb3, batch2d,
                         comp_features, p['cmp_W'].astype(F32), cmp_gb,
                         p['fc1_W'].astype(F32), fc1_gb,
                         p['fc2_W'].astype(F32), fc2_gb,
                         p['out_W'].astype(F32),
                         p['out_b'].reshape(1, 1).astype(F32))
    return out


# speedup vs baseline: 1.1574x; 1.0401x over previous
import functools

import jax
import jax.numpy as jnp
from jax import lax
from jax.experimental import pallas as pl
from jax.experimental.pallas import tpu as pltpu
from jax.experimental.pallas import tpu_sc as plsc

N = 10000
E = 320000
D_IN = 128
D_E = 16
H = 128
G = 64
COMP = 71

NC = 2
NS = 16
NW = NC * NS
EPW = E // NW

GB = 80
SB = 80
RPT = 632
NP = RPT * NS

EPS = 1e-5
F32 = jnp.float32


def _sp(x):
    return jnp.maximum(x, 0.0) + jnp.log1p(jnp.exp(-jnp.abs(x)))


def _scale_shift(ssum, ssq, count, g, beta):
    mean = ssum / count
    var = ssq / count - mean * mean
    scale = g / jnp.sqrt(var + EPS)
    shift = beta - mean * scale
    return jnp.stack([scale, shift]).reshape(2, H).astype(F32)


def _node_enc_body(x_ref, w_ref, gb_ref, wa_ref, wc_ref, h_ref, ha_ref, hc_ref):
    y = jnp.dot(x_ref[...], w_ref[...], preferred_element_type=F32)
    m = jnp.mean(y, axis=0, keepdims=True)
    v = jnp.mean((y - m) * (y - m), axis=0, keepdims=True)
    h = _sp(gb_ref[0:1, :] * (y - m) / jnp.sqrt(v + EPS) + gb_ref[1:2, :])
    h_ref[...] = h
    ha_ref[...] = jnp.dot(h, wa_ref[...], preferred_element_type=F32)
    hc_ref[...] = jnp.dot(h, wc_ref[...], preferred_element_type=F32)


def _node_enc(x, w, gb, wa, wc):
    return pl.pallas_call(
        _node_enc_body,
        out_shape=[jax.ShapeDtypeStruct((N, H), F32)] * 3,
    )(x, w, gb, wa, wc)


_GRAM_B = 4000


def _edge_stats_body(a_ref, w_ref, st_ref):
    i = pl.program_id(0)
    y = jnp.dot(a_ref[...], w_ref[...], preferred_element_type=F32)

    @pl.when(i == 0)
    def _():
        st_ref[...] = jnp.zeros_like(st_ref)

    st_ref[0:1, :] = st_ref[0:1, :] + jnp.sum(y, axis=0, keepdims=True)
    st_ref[1:2, :] = st_ref[1:2, :] + jnp.sum(y * y, axis=0, keepdims=True)


def _edge_stats(edge_attr, w):
    return pl.pallas_call(
        _edge_stats_body,
        grid=(E // _GRAM_B,),
        in_specs=[pl.BlockSpec((_GRAM_B, D_E), lambda i: (i, 0)),
                  pl.BlockSpec((D_E, H), lambda i: (0, 0))],
        out_specs=pl.BlockSpec((8, H), lambda i: (0, 0)),
        out_shape=jax.ShapeDtypeStruct((8, H), F32),
    )(edge_attr, w)


_EAB = 2000


def _edge_apply_body(a_ref, w_ref, ss_ref, wb0_ref, wb1_ref, wb2_ref, wb3_ref,
                     o0_ref, o1_ref, o2_ref, o3_ref):
    y = jnp.dot(a_ref[...], w_ref[...], preferred_element_type=F32)
    e = _sp(y * ss_ref[0:1, :] + ss_ref[1:2, :])
    o0_ref[...] = jnp.dot(e, wb0_ref[...], preferred_element_type=F32)
    o1_ref[...] = jnp.dot(e, wb1_ref[...], preferred_element_type=F32)
    o2_ref[...] = jnp.dot(e, wb2_ref[...], preferred_element_type=F32)
    o3_ref[...] = jnp.dot(e, wb3_ref[...], preferred_element_type=F32)


def _edge_apply(edge_attr, w, ss, wbs):
    blk = pl.BlockSpec((_EAB, H), lambda i: (i, 0))
    full = lambda shape: pl.BlockSpec(shape, lambda i: (0, 0))
    return pl.pallas_call(
        _edge_apply_body,
        grid=(E // _EAB,),
        in_specs=[pl.BlockSpec((_EAB, D_E), lambda i: (i, 0)),
                  full((D_E, H)), full((2, H)),
                  full((H, H)), full((H, H)), full((H, H)), full((H, H))],
        out_specs=[blk, blk, blk, blk],
        out_shape=[jax.ShapeDtypeStruct((E, H), F32)] * 4,
    )(edge_attr, w, ss, *wbs)


_P2B = 2000


def _p2s_body(y1_ref, ss_ref, w_ref, st_ref):
    i = pl.program_id(0)
    m1 = _sp(y1_ref[...] * ss_ref[0:1, :] + ss_ref[1:2, :])
    y2 = jnp.dot(m1, w_ref[...], preferred_element_type=F32)

    @pl.when(i == 0)
    def _():
        st_ref[...] = jnp.zeros_like(st_ref)

    st_ref[0:1, :] = st_ref[0:1, :] + jnp.sum(y2, axis=0, keepdims=True)
    st_ref[1:2, :] = st_ref[1:2, :] + jnp.sum(y2 * y2, axis=0, keepdims=True)


def _p2s(y1, ss, w):
    blk = pl.BlockSpec((_P2B, H), lambda i: (i, 0))
    full = lambda shape: pl.BlockSpec(shape, lambda i: (0, 0))
    return pl.pallas_call(
        _p2s_body,
        grid=(E // _P2B,),
        in_specs=[blk, full((2, H)), full((H, H))],
        out_specs=full((8, H)),
        out_shape=jax.ShapeDtypeStruct((8, H), F32),
    )(y1, ss, w)


def _p2a_body(y1_ref, ss1_ref, ss2_ref, w_ref, m_ref):
    m1 = _sp(y1_ref[...] * ss1_ref[0:1, :] + ss1_ref[1:2, :])
    y2 = jnp.dot(m1, w_ref[...], preferred_element_type=F32)
    m_ref[...] = _sp(y2 * ss2_ref[0:1, :] + ss2_ref[1:2, :])


def _p2a(y1, ss1, ss2, w):
    blk = pl.BlockSpec((_P2B, H), lambda i: (i, 0))
    full = lambda shape: pl.BlockSpec(shape, lambda i: (0, 0))
    return pl.pallas_call(
        _p2a_body,
        grid=(E // _P2B,),
        in_specs=[blk, full((2, H)), full((2, H)), full((H, H))],
        out_specs=blk,
        out_shape=jax.ShapeDtypeStruct((E, H), F32),
    )(y1, ss1, ss2, w)


def _p4_body(h_ref, ag_ref, w_ref, gb_ref, wa_ref, wc_ref,
             h2_ref, ha_ref, hc_ref):
    hin = h_ref[...] + ag_ref[0:N, :] + ag_ref[NP:NP + N, :]
    z = jnp.dot(hin, w_ref[...], preferred_element_type=F32)
    m = jnp.mean(z, axis=0, keepdims=True)
    v = jnp.mean((z - m) * (z - m), axis=0, keepdims=True)
    h2 = _sp(gb_ref[0:1, :] * (z - m) / jnp.sqrt(v + EPS) + gb_ref[1:2, :])
    h2_ref[...] = h2
    ha_ref[...] = jnp.dot(h2, wa_ref[...], preferred_element_type=F32)
    hc_ref[...] = jnp.dot(h2, wc_ref[...], preferred_element_type=F32)


def _p4(h, aggp, w, gb, wa, wc):
    return pl.pallas_call(
        _p4_body,
        out_shape=[jax.ShapeDtypeStruct((N, H), F32)] * 3,
    )(h, aggp, w, gb, wa, wc)


def _final_body(h_ref, ag_ref, w3_ref, gb3_ref, batch_ref, comp_ref,
                cmpw_ref, cmpgb_ref, fc1w_ref, fc1gb_ref, fc2w_ref, fc2gb_ref,
                outw_ref, outb_ref, o_ref):
    hin = h_ref[...] + ag_ref[0:N, :] + ag_ref[NP:NP + N, :]
    z = jnp.dot(hin, w3_ref[...], preferred_element_type=F32)
    m = jnp.mean(z, axis=0, keepdims=True)
    v = jnp.mean((z - m) * (z - m), axis=0, keepdims=True)
    h4 = _sp(gb3_ref[0:1, :] * (z - m) / jnp.sqrt(v + EPS) + gb3_ref[1:2, :])

    seg = jax.lax.broadcasted_iota(jnp.int32, (N, G), 1)
    p = (batch_ref[...] == seg).astype(F32)
    sums = lax.dot_general(p, h4, (((0,), (0,)), ((), ())),
                           preferred_element_type=F32,
                           precision=lax.Precision.HIGHEST)
    cnt = jnp.sum(p, axis=0)[:, None]
    gmean = sums / jnp.maximum(cnt, 1.0)

    def bn_sp(y, gb):
        mm = jnp.mean(y, axis=0, keepdims=True)
        vv = jnp.mean((y - mm) * (y - mm), axis=0, keepdims=True)
        return _sp(gb[0:1, :] * (y - mm) / jnp.sqrt(vv + EPS) + gb[1:2, :])

    cf = bn_sp(jnp.dot(comp_ref[...], cmpw_ref[...],
                       preferred_element_type=F32), cmpgb_ref[...])
    g1 = jnp.concatenate([gmean, cf], axis=1)
    g2 = bn_sp(jnp.dot(g1, fc1w_ref[...], preferred_element_type=F32),
               fc1gb_ref[...])
    g3 = bn_sp(jnp.dot(g2, fc2w_ref[...], preferred_element_type=F32),
               fc2gb_ref[...])
    o_ref[...] = (jnp.dot(g3, outw_ref[...], preferred_element_type=F32)
                  + outb_ref[0:1, :])


def _final(h, aggp, w3, gb3, batch2d, comp, cmpw, cmpgb, fc1w, fc1gb,
           fc2w, fc2gb, outw, outb):
    return pl.pallas_call(
        _final_body,
        out_shape=jax.ShapeDtypeStruct((G, 1), F32),
    )(h, aggp, w3, gb3, batch2d, comp, cmpw, cmpgb, fc1w, fc1gb,
      fc2w, fc2gb, outw, outb)


@functools.cache
def _mesh():
    return plsc.VectorSubcoreMesh(core_axis_name="c", subcore_axis_name="s",
                                  num_cores=NC, num_subcores=NS)


_NCH = EPW // GB


def _gather_body(ha_hbm, hc_hbm, eb_hbm, row_hbm, col_hbm,
                 y1_hbm, stats_hbm,
                 idxr, idxc, bufa, bufc, bufe, bufy, sbuf, sems):
    cid = lax.axis_index("c")
    sid = lax.axis_index("s")
    wid = sid * NC + cid
    base0 = wid * EPW

    pltpu.sync_copy(row_hbm.at[wid], idxr)
    pltpu.sync_copy(col_hbm.at[wid], idxc)

    def issue(c, b):
        base = base0 + c * GB
        isl = pl.ds(c * GB, GB)
        pltpu.async_copy(ha_hbm.at[idxr.at[isl]], bufa.at[b], sems.at[b, 0])
        pltpu.async_copy(hc_hbm.at[idxc.at[isl]], bufc.at[b], sems.at[b, 1])
        pltpu.async_copy(eb_hbm.at[pl.ds(base, GB)], bufe.at[b], sems.at[b, 2])

    def wait_in(b):
        pltpu.make_async_copy(eb_hbm.at[pl.ds(0, GB)], bufa.at[b],
                              sems.at[b, 0]).wait()
        pltpu.make_async_copy(eb_hbm.at[pl.ds(0, GB)], bufc.at[b],
                              sems.at[b, 1]).wait()
        pltpu.make_async_copy(eb_hbm.at[pl.ds(0, GB)], bufe.at[b],
                              sems.at[b, 2]).wait()

    def wait_wb(b):
        pltpu.make_async_copy(eb_hbm.at[pl.ds(0, GB)], bufy.at[b],
                              sems.at[b, 3]).wait()

    def compute(c, b, accs):
        def rowloop(r, acc):
            acc_s, acc_q = acc
            ns, nq = [], []
            for k in range(H // 16):
                sl = pl.ds(16 * k, 16)
                y = bufa[b, r, sl] + bufc[b, r, sl] + bufe[b, r, sl]
                bufy[b, r, sl] = y
                ns.append(acc_s[k] + y)
                nq.append(acc_q[k] + y * y)
            return (tuple(ns), tuple(nq))

        accs = lax.fori_loop(0, GB, rowloop, accs)
        pltpu.async_copy(bufy.at[b], y1_hbm.at[pl.ds(base0 + c * GB, GB)],
                         sems.at[b, 3])
        return accs

    z = jnp.zeros((16,), F32)
    accs = (tuple(z for _ in range(H // 16)), tuple(z for _ in range(H // 16)))

    issue(0, 0)
    issue(1, 1)

    def outer(k2, accs):
        for j in range(2):
            c = 2 * k2 + j
            wait_in(j)

            @pl.when(k2 > 0)
            def _():
                wait_wb(j)

            accs = compute(c, j, accs)

            @pl.when(c + 2 < _NCH)
            def _():
                issue(c + 2, j)
        return accs

    accs = lax.fori_loop(0, (_NCH - 1) // 2, outer, accs)
    wait_in(0)
    wait_wb(0)
    accs = compute(_NCH - 1, 0, accs)
    wait_wb(1)
    wait_wb(0)

    acc_s, acc_q = accs
    for k in range(H // 16):
        sl = pl.ds(16 * k, 16)
        sbuf[0, sl] = acc_s[k]
        sbuf[1, sl] = acc_q[k]
    pltpu.sync_copy(sbuf, stats_hbm.at[pl.ds(wid * 8, 8)])


@functools.cache
def _gather_call():
    return pl.kernel(
        _gather_body,
        out_type=[jax.ShapeDtypeStruct((E, H), F32),
                  jax.ShapeDtypeStruct((NW * 8, H), F32)],
        mesh=_mesh(),
        scratch_types=[
            pltpu.VMEM((EPW,), jnp.int32),
            pltpu.VMEM((EPW,), jnp.int32),
            pltpu.VMEM((2, GB, H), F32),
            pltpu.VMEM((2, GB, H), F32),
            pltpu.VMEM((2, GB, H), F32),
            pltpu.VMEM((2, GB, H), F32),
            pltpu.VMEM((8, H), F32),
            pltpu.SemaphoreType.DMA((2, 4)),
        ],
    )


_NCHS = EPW // SB


def _scatter_body(m_hbm, col_hbm, z_hbm, aggp_hbm, shared, idxall, mbuf, sems):
    cid = lax.axis_index("c")
    sid = lax.axis_index("s")
    wid = sid * NC + cid
    base0 = wid * EPW
    pltpu.sync_copy(col_hbm.at[wid], idxall)
    pltpu.sync_copy(z_hbm.at[pl.ds(sid * RPT, RPT)],
                    shared.at[pl.ds(sid * RPT, RPT)])
    plsc.subcore_barrier()

    def issue(c, b):
        pltpu.async_copy(m_hbm.at[pl.ds(base0 + c * SB, SB)], mbuf.at[b],
                         sems.at[b])

    def wait_in(b):
        pltpu.make_async_copy(m_hbm.at[pl.ds(0, SB)], mbuf.at[b],
                              sems.at[b]).wait()

    issue(0, 0)
    issue(1, 1)

    def outer(k2, carry):
        for j in range(2):
            c = 2 * k2 + j
            wait_in(j)
            pltpu.sync_copy(mbuf.at[j], shared.at[idxall.at[c]], add=True)

            @pl.when(c + 2 < _NCHS)
            def _():
                issue(c + 2, j)
        return carry

    lax.fori_loop(0, (_NCHS - 1) // 2, outer, 0)
    wait_in(0)
    pltpu.sync_copy(mbuf.at[0], shared.at[idxall.at[_NCHS - 1]], add=True)
    plsc.subcore_barrier()
    pltpu.sync_copy(shared.at[pl.ds(sid * RPT, RPT)],
                    aggp_hbm.at[pl.ds(cid * NP + sid * RPT, RPT)])


@functools.cache
def _scatter_call():
    return pl.kernel(
        _scatter_body,
        out_type=jax.ShapeDtypeStruct((NC * NP, H), F32),
        mesh=_mesh(),
        scratch_types=[
            pltpu.VMEM_SHARED((NP, H), F32),
            pltpu.VMEM((_NCHS, SB), jnp.int32),
            pltpu.VMEM((2, SB, H), F32),
            pltpu.SemaphoreType.DMA((2,)),
        ],
    )


def kernel(x, edge_attr, comp_features, params, edge_index, batch):
    p = params
    x = x.astype(F32)
    edge_attr = edge_attr.astype(F32)
    comp_features = comp_features.astype(F32)
    row = edge_index[0].astype(jnp.int32)
    col = edge_index[1].astype(jnp.int32)
    row2 = row.reshape(NW, EPW)
    col2 = col.reshape(NW, EPW)
    col3 = col.reshape(NW, _NCHS, SB)
    batch2d = batch.astype(jnp.int32).reshape(N, 1)
    zeros_n = jnp.zeros((NP, H), F32)

    convs = p['convs']
    was = [c['e1_W'][0:H].astype(F32) for c in convs]
    wbs = [c['e1_W'][H:2 * H].astype(F32) for c in convs]
    wcs = [c['e1_W'][2 * H:3 * H].astype(F32) for c in convs]

    def gb(gname, bname, src):
        return jnp.stack([src[gname], src[bname]]).reshape(2, H).astype(F32)

    ne_gb = jnp.stack([p['ne_g'], p['ne_beta']]).reshape(2, H).astype(F32)
    h, ha, hc = _node_enc(x, p['ne_W'].astype(F32), ne_gb, was[0], wcs[0])

    w_ee = p['ee_W'].astype(F32)
    est = _edge_stats(edge_attr, w_ee)
    mean_y = est[0] / E
    var_y = est[1] / E - mean_y * mean_y
    scale = p['ee_g'] / jnp.sqrt(var_y + EPS)
    shift = p['ee_beta'] - mean_y * scale
    ee_ss = jnp.stack([scale, shift]).reshape(2, H).astype(F32)
    ebs = _edge_apply(edge_attr, w_ee, ee_ss, wbs)

    for l in range(4):
        c = convs[l]
        y1, stats = _gather_call()(ha, hc, ebs[l], row2, col2)
        st = stats.reshape(NW, 8, H)
        ss1 = _scale_shift(jnp.sum(st[:, 0], axis=0), jnp.sum(st[:, 1], axis=0),
                           float(E), c['bn1_g'], c['bn1_b'])
        w2 = c['e2_W'].astype(F32)
        st2 = _p2s(y1, ss1, w2)
        ss2 = _scale_shift(st2[0], st2[1], float(E), c['bn2_g'], c['bn2_b'])
        msg = _p2a(y1, ss1, ss2, w2)
        aggp = _scatter_call()(msg, col3, zeros_n)
        gb3 = gb('bn3_g', 'bn3_b', c)
        if l < 3:
            h, ha, hc = _p4(h, aggp, c['n_W'].astype(F32), gb3,
                            was[l + 1], wcs[l + 1])
        else:
            cmp_gb = gb('cmp_g', 'cmp_beta', p)
            fc1_gb = gb('fc1_g', 'fc1_beta', p)
            fc2_gb = gb('fc2_g', 'fc2_beta', p)
            out = _final(h, aggp, c['n_W'].astype(F32), gb3, batch2d,
                         comp_features, p['cmp_W'].astype(F32), cmp_gb,
                         p['fc1_W'].astype(F32), fc1_gb,
                         p['fc2_W'].astype(F32), fc2_gb,
                         p['out_W'].astype(F32),
                         p['out_b'].reshape(1, 1).astype(F32))
    return out


# speedup vs baseline: 1.2776x; 1.1039x over previous
import functools

import jax
import jax.numpy as jnp
from jax import lax
from jax.experimental import pallas as pl
from jax.experimental.pallas import tpu as pltpu
from jax.experimental.pallas import tpu_sc as plsc

N = 10000
E = 320000
D_IN = 128
D_E = 16
H = 128
G = 64
COMP = 71

NC = 2
NS = 16
NW = NC * NS
EPW = E // NW

GB = 80
SB = 80
RPT = 632
NP = RPT * NS

EPS = 1e-5
F32 = jnp.float32


def _sp(x):
    return jnp.maximum(x, 0.0) + jnp.log1p(jnp.exp(-jnp.abs(x)))


def _scale_shift(ssum, ssq, count, g, beta):
    mean = ssum / count
    var = ssq / count - mean * mean
    scale = g / jnp.sqrt(var + EPS)
    shift = beta - mean * scale
    return jnp.stack([scale, shift]).reshape(2, H).astype(F32)


def _node_enc_body(x_ref, w_ref, gb_ref, wa_ref, wc_ref, h_ref, ha_ref, hc_ref):
    y = jnp.dot(x_ref[...], w_ref[...], preferred_element_type=F32)
    m = jnp.mean(y, axis=0, keepdims=True)
    v = jnp.mean((y - m) * (y - m), axis=0, keepdims=True)
    h = _sp(gb_ref[0:1, :] * (y - m) / jnp.sqrt(v + EPS) + gb_ref[1:2, :])
    h_ref[...] = h
    ha_ref[...] = jnp.dot(h, wa_ref[...], preferred_element_type=F32)
    hc_ref[...] = jnp.dot(h, wc_ref[...], preferred_element_type=F32)


def _node_enc(x, w, gb, wa, wc):
    return pl.pallas_call(
        _node_enc_body,
        out_shape=[jax.ShapeDtypeStruct((N, H), F32)] * 3,
    )(x, w, gb, wa, wc)


_GRAM_B = 4000


def _edge_stats_body(a_ref, w_ref, st_ref):
    i = pl.program_id(0)
    y = jnp.dot(a_ref[...], w_ref[...], preferred_element_type=F32)

    @pl.when(i == 0)
    def _():
        st_ref[...] = jnp.zeros_like(st_ref)

    st_ref[0:1, :] = st_ref[0:1, :] + jnp.sum(y, axis=0, keepdims=True)
    st_ref[1:2, :] = st_ref[1:2, :] + jnp.sum(y * y, axis=0, keepdims=True)


def _edge_stats(edge_attr, w):
    return pl.pallas_call(
        _edge_stats_body,
        grid=(E // _GRAM_B,),
        in_specs=[pl.BlockSpec((_GRAM_B, D_E), lambda i: (i, 0)),
                  pl.BlockSpec((D_E, H), lambda i: (0, 0))],
        out_specs=pl.BlockSpec((8, H), lambda i: (0, 0)),
        out_shape=jax.ShapeDtypeStruct((8, H), F32),
    )(edge_attr, w)


_EAB = 4000


def _edge_apply_body(a_ref, w_ref, ss_ref, wb0_ref, wb1_ref, wb2_ref, wb3_ref,
                     o0_ref, o1_ref, o2_ref, o3_ref):
    y = jnp.dot(a_ref[...], w_ref[...], preferred_element_type=F32)
    e = _sp(y * ss_ref[0:1, :] + ss_ref[1:2, :])
    o0_ref[...] = jnp.dot(e, wb0_ref[...], preferred_element_type=F32)
    o1_ref[...] = jnp.dot(e, wb1_ref[...], preferred_element_type=F32)
    o2_ref[...] = jnp.dot(e, wb2_ref[...], preferred_element_type=F32)
    o3_ref[...] = jnp.dot(e, wb3_ref[...], preferred_element_type=F32)


def _edge_apply(edge_attr, w, ss, wbs):
    blk = pl.BlockSpec((_EAB, H), lambda i: (i, 0))
    full = lambda shape: pl.BlockSpec(shape, lambda i: (0, 0))
    return pl.pallas_call(
        _edge_apply_body,
        grid=(E // _EAB,),
        in_specs=[pl.BlockSpec((_EAB, D_E), lambda i: (i, 0)),
                  full((D_E, H)), full((2, H)),
                  full((H, H)), full((H, H)), full((H, H)), full((H, H))],
        out_specs=[blk, blk, blk, blk],
        out_shape=[jax.ShapeDtypeStruct((E, H), F32)] * 4,
    )(edge_attr, w, ss, *wbs)


_P2B = 4000


def _p2s_body(y1_ref, ss_ref, w_ref, st_ref):
    i = pl.program_id(0)
    m1 = _sp(y1_ref[...] * ss_ref[0:1, :] + ss_ref[1:2, :])
    y2 = jnp.dot(m1, w_ref[...], preferred_element_type=F32)

    @pl.when(i == 0)
    def _():
        st_ref[...] = jnp.zeros_like(st_ref)

    st_ref[0:1, :] = st_ref[0:1, :] + jnp.sum(y2, axis=0, keepdims=True)
    st_ref[1:2, :] = st_ref[1:2, :] + jnp.sum(y2 * y2, axis=0, keepdims=True)


def _p2s(y1, ss, w):
    blk = pl.BlockSpec((_P2B, H), lambda i: (i, 0))
    full = lambda shape: pl.BlockSpec(shape, lambda i: (0, 0))
    return pl.pallas_call(
        _p2s_body,
        grid=(E // _P2B,),
        in_specs=[blk, full((2, H)), full((H, H))],
        out_specs=full((8, H)),
        out_shape=jax.ShapeDtypeStruct((8, H), F32),
    )(y1, ss, w)


def _p2a_body(y1_ref, ss1_ref, ss2_ref, w_ref, m_ref):
    m1 = _sp(y1_ref[...] * ss1_ref[0:1, :] + ss1_ref[1:2, :])
    y2 = jnp.dot(m1, w_ref[...], preferred_element_type=F32)
    m_ref[...] = _sp(y2 * ss2_ref[0:1, :] + ss2_ref[1:2, :])


def _p2a(y1, ss1, ss2, w):
    blk = pl.BlockSpec((_P2B, H), lambda i: (i, 0))
    full = lambda shape: pl.BlockSpec(shape, lambda i: (0, 0))
    return pl.pallas_call(
        _p2a_body,
        grid=(E // _P2B,),
        in_specs=[blk, full((2, H)), full((2, H)), full((H, H))],
        out_specs=blk,
        out_shape=jax.ShapeDtypeStruct((E, H), F32),
    )(y1, ss1, ss2, w)


def _p4_body(h_ref, ag_ref, w_ref, gb_ref, wa_ref, wc_ref,
             h2_ref, ha_ref, hc_ref):
    hin = h_ref[...] + ag_ref[0:N, :] + ag_ref[NP:NP + N, :]
    z = jnp.dot(hin, w_ref[...], preferred_element_type=F32)
    m = jnp.mean(z, axis=0, keepdims=True)
    v = jnp.mean((z - m) * (z - m), axis=0, keepdims=True)
    h2 = _sp(gb_ref[0:1, :] * (z - m) / jnp.sqrt(v + EPS) + gb_ref[1:2, :])
    h2_ref[...] = h2
    ha_ref[...] = jnp.dot(h2, wa_ref[...], preferred_element_type=F32)
    hc_ref[...] = jnp.dot(h2, wc_ref[...], preferred_element_type=F32)


def _p4(h, aggp, w, gb, wa, wc):
    return pl.pallas_call(
        _p4_body,
        out_shape=[jax.ShapeDtypeStruct((N, H), F32)] * 3,
    )(h, aggp, w, gb, wa, wc)


def _final_body(h_ref, ag_ref, w3_ref, gb3_ref, batch_ref, comp_ref,
                cmpw_ref, cmpgb_ref, fc1w_ref, fc1gb_ref, fc2w_ref, fc2gb_ref,
                outw_ref, outb_ref, o_ref):
    hin = h_ref[...] + ag_ref[0:N, :] + ag_ref[NP:NP + N, :]
    z = jnp.dot(hin, w3_ref[...], preferred_element_type=F32)
    m = jnp.mean(z, axis=0, keepdims=True)
    v = jnp.mean((z - m) * (z - m), axis=0, keepdims=True)
    h4 = _sp(gb3_ref[0:1, :] * (z - m) / jnp.sqrt(v + EPS) + gb3_ref[1:2, :])

    seg = jax.lax.broadcasted_iota(jnp.int32, (N, G), 1)
    p = (batch_ref[...] == seg).astype(F32)
    sums = lax.dot_general(p, h4, (((0,), (0,)), ((), ())),
                           preferred_element_type=F32,
                           precision=lax.Precision.HIGHEST)
    cnt = jnp.sum(p, axis=0)[:, None]
    gmean = sums / jnp.maximum(cnt, 1.0)

    def bn_sp(y, gb):
        mm = jnp.mean(y, axis=0, keepdims=True)
        vv = jnp.mean((y - mm) * (y - mm), axis=0, keepdims=True)
        return _sp(gb[0:1, :] * (y - mm) / jnp.sqrt(vv + EPS) + gb[1:2, :])

    cf = bn_sp(jnp.dot(comp_ref[...], cmpw_ref[...],
                       preferred_element_type=F32), cmpgb_ref[...])
    g1 = jnp.concatenate([gmean, cf], axis=1)
    g2 = bn_sp(jnp.dot(g1, fc1w_ref[...], preferred_element_type=F32),
               fc1gb_ref[...])
    g3 = bn_sp(jnp.dot(g2, fc2w_ref[...], preferred_element_type=F32),
               fc2gb_ref[...])
    o_ref[...] = (jnp.dot(g3, outw_ref[...], preferred_element_type=F32)
                  + outb_ref[0:1, :])


def _final(h, aggp, w3, gb3, batch2d, comp, cmpw, cmpgb, fc1w, fc1gb,
           fc2w, fc2gb, outw, outb):
    return pl.pallas_call(
        _final_body,
        out_shape=jax.ShapeDtypeStruct((G, 1), F32),
    )(h, aggp, w3, gb3, batch2d, comp, cmpw, cmpgb, fc1w, fc1gb,
      fc2w, fc2gb, outw, outb)


@functools.cache
def _mesh():
    return plsc.VectorSubcoreMesh(core_axis_name="c", subcore_axis_name="s",
                                  num_cores=NC, num_subcores=NS)


_NCH = EPW // GB


def _gather_body(ha_hbm, hc_hbm, eb_hbm, row_hbm, col_hbm,
                 y1_hbm, stats_hbm,
                 idxr, idxc, bufa, bufc, bufe, bufy, sbuf, sems):
    cid = lax.axis_index("c")
    sid = lax.axis_index("s")
    wid = sid * NC + cid
    base0 = wid * EPW

    pltpu.sync_copy(row_hbm.at[wid], idxr)
    pltpu.sync_copy(col_hbm.at[wid], idxc)

    def issue(c, b):
        base = base0 + c * GB
        isl = pl.ds(c * GB, GB)
        pltpu.async_copy(ha_hbm.at[idxr.at[isl]], bufa.at[b], sems.at[b, 0])
        pltpu.async_copy(hc_hbm.at[idxc.at[isl]], bufc.at[b], sems.at[b, 1])
        pltpu.async_copy(eb_hbm.at[pl.ds(base, GB)], bufe.at[b], sems.at[b, 2])

    def wait_in(b):
        pltpu.make_async_copy(eb_hbm.at[pl.ds(0, GB)], bufa.at[b],
                              sems.at[b, 0]).wait()
        pltpu.make_async_copy(eb_hbm.at[pl.ds(0, GB)], bufc.at[b],
                              sems.at[b, 1]).wait()
        pltpu.make_async_copy(eb_hbm.at[pl.ds(0, GB)], bufe.at[b],
                              sems.at[b, 2]).wait()

    def wait_wb(b):
        pltpu.make_async_copy(eb_hbm.at[pl.ds(0, GB)], bufy.at[b],
                              sems.at[b, 3]).wait()

    def compute(c, b, accs):
        def rowloop(r, acc):
            acc_s, acc_q = acc
            ns, nq = [], []
            for k in range(H // 16):
                sl = pl.ds(16 * k, 16)
                y = bufa[b, r, sl] + bufc[b, r, sl] + bufe[b, r, sl]
                bufy[b, r, sl] = y
                ns.append(acc_s[k] + y)
                nq.append(acc_q[k] + y * y)
            return (tuple(ns), tuple(nq))

        accs = lax.fori_loop(0, GB, rowloop, accs)
        pltpu.async_copy(bufy.at[b], y1_hbm.at[pl.ds(base0 + c * GB, GB)],
                         sems.at[b, 3])
        return accs

    z = jnp.zeros((16,), F32)
    accs = (tuple(z for _ in range(H // 16)), tuple(z for _ in range(H // 16)))

    issue(0, 0)
    issue(1, 1)

    def outer(k2, accs):
        for j in range(2):
            c = 2 * k2 + j
            wait_in(j)

            @pl.when(k2 > 0)
            def _():
                wait_wb(j)

            accs = compute(c, j, accs)

            @pl.when(c + 2 < _NCH)
            def _():
                issue(c + 2, j)
        return accs

    accs = lax.fori_loop(0, (_NCH - 1) // 2, outer, accs)
    wait_in(0)
    wait_wb(0)
    accs = compute(_NCH - 1, 0, accs)
    wait_wb(1)
    wait_wb(0)

    acc_s, acc_q = accs
    for k in range(H // 16):
        sl = pl.ds(16 * k, 16)
        sbuf[0, sl] = acc_s[k]
        sbuf[1, sl] = acc_q[k]
    pltpu.sync_copy(sbuf, stats_hbm.at[pl.ds(wid * 8, 8)])


@functools.cache
def _gather_call():
    return pl.kernel(
        _gather_body,
        out_type=[jax.ShapeDtypeStruct((E, H), F32),
                  jax.ShapeDtypeStruct((NW * 8, H), F32)],
        mesh=_mesh(),
        scratch_types=[
            pltpu.VMEM((EPW,), jnp.int32),
            pltpu.VMEM((EPW,), jnp.int32),
            pltpu.VMEM((2, GB, H), F32),
            pltpu.VMEM((2, GB, H), F32),
            pltpu.VMEM((2, GB, H), F32),
            pltpu.VMEM((2, GB, H), F32),
            pltpu.VMEM((8, H), F32),
            pltpu.SemaphoreType.DMA((2, 4)),
        ],
    )


_NCHS = EPW // SB


def _scatter_body(m_hbm, col_hbm, z_hbm, aggp_hbm, shared, idxall, mbuf, sems):
    cid = lax.axis_index("c")
    sid = lax.axis_index("s")
    wid = sid * NC + cid
    base0 = wid * EPW
    pltpu.sync_copy(col_hbm.at[wid], idxall)
    pltpu.sync_copy(z_hbm.at[pl.ds(sid * RPT, RPT)],
                    shared.at[pl.ds(sid * RPT, RPT)])
    plsc.subcore_barrier()

    def issue(c, b):
        pltpu.async_copy(m_hbm.at[pl.ds(base0 + c * SB, SB)], mbuf.at[b],
                         sems.at[b])

    def wait_in(b):
        pltpu.make_async_copy(m_hbm.at[pl.ds(0, SB)], mbuf.at[b],
                              sems.at[b]).wait()

    issue(0, 0)
    issue(1, 1)

    def outer(k2, carry):
        for j in range(2):
            c = 2 * k2 + j
            wait_in(j)
            pltpu.sync_copy(mbuf.at[j], shared.at[idxall.at[c]], add=True)

            @pl.when(c + 2 < _NCHS)
            def _():
                issue(c + 2, j)
        return carry

    lax.fori_loop(0, (_NCHS - 1) // 2, outer, 0)
    wait_in(0)
    pltpu.sync_copy(mbuf.at[0], shared.at[idxall.at[_NCHS - 1]], add=True)
    plsc.subcore_barrier()
    pltpu.sync_copy(shared.at[pl.ds(sid * RPT, RPT)],
                    aggp_hbm.at[pl.ds(cid * NP + sid * RPT, RPT)])


@functools.cache
def _scatter_call():
    return pl.kernel(
        _scatter_body,
        out_type=jax.ShapeDtypeStruct((NC * NP, H), F32),
        mesh=_mesh(),
        scratch_types=[
            pltpu.VMEM_SHARED((NP, H), F32),
            pltpu.VMEM((_NCHS, SB), jnp.int32),
            pltpu.VMEM((2, SB, H), F32),
            pltpu.SemaphoreType.DMA((2,)),
        ],
    )


def kernel(x, edge_attr, comp_features, params, edge_index, batch):
    p = params
    x = x.astype(F32)
    edge_attr = edge_attr.astype(F32)
    comp_features = comp_features.astype(F32)
    row = edge_index[0].astype(jnp.int32)
    col = edge_index[1].astype(jnp.int32)
    row2 = row.reshape(NW, EPW)
    col2 = col.reshape(NW, EPW)
    col3 = col.reshape(NW, _NCHS, SB)
    batch2d = batch.astype(jnp.int32).reshape(N, 1)
    zeros_n = jnp.zeros((NP, H), F32)

    convs = p['convs']
    was = [c['e1_W'][0:H].astype(F32) for c in convs]
    wbs = [c['e1_W'][H:2 * H].astype(F32) for c in convs]
    wcs = [c['e1_W'][2 * H:3 * H].astype(F32) for c in convs]

    def gb(gname, bname, src):
        return jnp.stack([src[gname], src[bname]]).reshape(2, H).astype(F32)

    ne_gb = jnp.stack([p['ne_g'], p['ne_beta']]).reshape(2, H).astype(F32)
    h, ha, hc = _node_enc(x, p['ne_W'].astype(F32), ne_gb, was[0], wcs[0])

    w_ee = p['ee_W'].astype(F32)
    est = _edge_stats(edge_attr, w_ee)
    mean_y = est[0] / E
    var_y = est[1] / E - mean_y * mean_y
    scale = p['ee_g'] / jnp.sqrt(var_y + EPS)
    shift = p['ee_beta'] - mean_y * scale
    ee_ss = jnp.stack([scale, shift]).reshape(2, H).astype(F32)
    ebs = _edge_apply(edge_attr, w_ee, ee_ss, wbs)

    for l in range(4):
        c = convs[l]
        y1, stats = _gather_call()(ha, hc, ebs[l], row2, col2)
        st = stats.reshape(NW, 8, H)
        ss1 = _scale_shift(jnp.sum(st[:, 0], axis=0), jnp.sum(st[:, 1], axis=0),
                           float(E), c['bn1_g'], c['bn1_b'])
        w2 = c['e2_W'].astype(F32)
        st2 = _p2s(y1, ss1, w2)
        ss2 = _scale_shift(st2[0], st2[1], float(E), c['bn2_g'], c['bn2_b'])
        msg = _p2a(y1, ss1, ss2, w2)
        aggp = _scatter_call()(msg, col3, zeros_n)
        gb3 = gb('bn3_g', 'bn3_b', c)
        if l < 3:
            h, ha, hc = _p4(h, aggp, c['n_W'].astype(F32), gb3,
                            was[l + 1], wcs[l + 1])
        else:
            cmp_gb = gb('cmp_g', 'cmp_beta', p)
            fc1_gb = gb('fc1_g', 'fc1_beta', p)
            fc2_gb = gb('fc2_g', 'fc2_beta', p)
            out = _final(h, aggp, c['n_W'].astype(F32), gb3, batch2d,
                         comp_features, p['cmp_W'].astype(F32), cmp_gb,
                         p['fc1_W'].astype(F32), fc1_gb,
                         p['fc2_W'].astype(F32), fc2_gb,
                         p['out_W'].astype(F32),
                         p['out_b'].reshape(1, 1).astype(F32))
    return out


# speedup vs baseline: 1.2976x; 1.0157x over previous
import functools

import jax
import jax.numpy as jnp
from jax import lax
from jax.experimental import pallas as pl
from jax.experimental.pallas import tpu as pltpu
from jax.experimental.pallas import tpu_sc as plsc

N = 10000
E = 320000
D_IN = 128
D_E = 16
H = 128
G = 64
COMP = 71

NC = 2
NS = 16
NW = NC * NS
EPW = E // NW

GB = 80
SB = 80
RPT = 632
NP = RPT * NS

EPS = 1e-5
F32 = jnp.float32


def _sp(x):
    return jnp.maximum(x, 0.0) + jnp.log1p(jnp.exp(-jnp.abs(x)))


def _scale_shift(ssum, ssq, count, g, beta):
    mean = ssum / count
    var = ssq / count - mean * mean
    scale = g / jnp.sqrt(var + EPS)
    shift = beta - mean * scale
    return jnp.stack([scale, shift]).reshape(2, H).astype(F32)


def _node_enc_body(x_ref, w_ref, gb_ref, wa_ref, wc_ref, h_ref, ha_ref, hc_ref):
    y = jnp.dot(x_ref[...], w_ref[...], preferred_element_type=F32)
    m = jnp.mean(y, axis=0, keepdims=True)
    v = jnp.mean((y - m) * (y - m), axis=0, keepdims=True)
    h = _sp(gb_ref[0:1, :] * (y - m) / jnp.sqrt(v + EPS) + gb_ref[1:2, :])
    h_ref[...] = h
    ha_ref[...] = jnp.dot(h, wa_ref[...], preferred_element_type=F32)
    hc_ref[...] = jnp.dot(h, wc_ref[...], preferred_element_type=F32)


def _node_enc(x, w, gb, wa, wc):
    return pl.pallas_call(
        _node_enc_body,
        out_shape=[jax.ShapeDtypeStruct((N, H), F32)] * 3,
    )(x, w, gb, wa, wc)


_GRAM_B = 4000


def _edge_stats_body(a_ref, w_ref, st_ref):
    i = pl.program_id(0)
    y = jnp.dot(a_ref[...], w_ref[...], preferred_element_type=F32)

    @pl.when(i == 0)
    def _():
        st_ref[...] = jnp.zeros_like(st_ref)

    st_ref[0:1, :] = st_ref[0:1, :] + jnp.sum(y, axis=0, keepdims=True)
    st_ref[1:2, :] = st_ref[1:2, :] + jnp.sum(y * y, axis=0, keepdims=True)


def _edge_stats(edge_attr, w):
    return pl.pallas_call(
        _edge_stats_body,
        grid=(E // _GRAM_B,),
        in_specs=[pl.BlockSpec((_GRAM_B, D_E), lambda i: (i, 0)),
                  pl.BlockSpec((D_E, H), lambda i: (0, 0))],
        out_specs=pl.BlockSpec((8, H), lambda i: (0, 0)),
        out_shape=jax.ShapeDtypeStruct((8, H), F32),
    )(edge_attr, w)


_EAB = 8000


def _edge_apply_body(a_ref, w_ref, ss_ref, wb0_ref, wb1_ref, wb2_ref, wb3_ref,
                     o0_ref, o1_ref, o2_ref, o3_ref):
    y = jnp.dot(a_ref[...], w_ref[...], preferred_element_type=F32)
    e = _sp(y * ss_ref[0:1, :] + ss_ref[1:2, :])
    o0_ref[...] = jnp.dot(e, wb0_ref[...], preferred_element_type=F32)
    o1_ref[...] = jnp.dot(e, wb1_ref[...], preferred_element_type=F32)
    o2_ref[...] = jnp.dot(e, wb2_ref[...], preferred_element_type=F32)
    o3_ref[...] = jnp.dot(e, wb3_ref[...], preferred_element_type=F32)


def _edge_apply(edge_attr, w, ss, wbs):
    blk = pl.BlockSpec((_EAB, H), lambda i: (i, 0))
    full = lambda shape: pl.BlockSpec(shape, lambda i: (0, 0))
    return pl.pallas_call(
        _edge_apply_body,
        grid=(E // _EAB,),
        in_specs=[pl.BlockSpec((_EAB, D_E), lambda i: (i, 0)),
                  full((D_E, H)), full((2, H)),
                  full((H, H)), full((H, H)), full((H, H)), full((H, H))],
        out_specs=[blk, blk, blk, blk],
        out_shape=[jax.ShapeDtypeStruct((E, H), F32)] * 4,
    )(edge_attr, w, ss, *wbs)


_P2B = 8000


def _p2s_body(y1_ref, ss_ref, w_ref, st_ref):
    i = pl.program_id(0)
    m1 = _sp(y1_ref[...] * ss_ref[0:1, :] + ss_ref[1:2, :])
    y2 = jnp.dot(m1, w_ref[...], preferred_element_type=F32)

    @pl.when(i == 0)
    def _():
        st_ref[...] = jnp.zeros_like(st_ref)

    st_ref[0:1, :] = st_ref[0:1, :] + jnp.sum(y2, axis=0, keepdims=True)
    st_ref[1:2, :] = st_ref[1:2, :] + jnp.sum(y2 * y2, axis=0, keepdims=True)


def _p2s(y1, ss, w):
    blk = pl.BlockSpec((_P2B, H), lambda i: (i, 0))
    full = lambda shape: pl.BlockSpec(shape, lambda i: (0, 0))
    return pl.pallas_call(
        _p2s_body,
        grid=(E // _P2B,),
        in_specs=[blk, full((2, H)), full((H, H))],
        out_specs=full((8, H)),
        out_shape=jax.ShapeDtypeStruct((8, H), F32),
    )(y1, ss, w)


def _p2a_body(y1_ref, ss1_ref, ss2_ref, w_ref, m_ref):
    m1 = _sp(y1_ref[...] * ss1_ref[0:1, :] + ss1_ref[1:2, :])
    y2 = jnp.dot(m1, w_ref[...], preferred_element_type=F32)
    m_ref[...] = _sp(y2 * ss2_ref[0:1, :] + ss2_ref[1:2, :])


def _p2a(y1, ss1, ss2, w):
    blk = pl.BlockSpec((_P2B, H), lambda i: (i, 0))
    full = lambda shape: pl.BlockSpec(shape, lambda i: (0, 0))
    return pl.pallas_call(
        _p2a_body,
        grid=(E // _P2B,),
        in_specs=[blk, full((2, H)), full((2, H)), full((H, H))],
        out_specs=blk,
        out_shape=jax.ShapeDtypeStruct((E, H), F32),
    )(y1, ss1, ss2, w)


def _p4_body(h_ref, ag_ref, w_ref, gb_ref, wa_ref, wc_ref,
             h2_ref, ha_ref, hc_ref):
    hin = h_ref[...] + ag_ref[0:N, :] + ag_ref[NP:NP + N, :]
    z = jnp.dot(hin, w_ref[...], preferred_element_type=F32)
    m = jnp.mean(z, axis=0, keepdims=True)
    v = jnp.mean((z - m) * (z - m), axis=0, keepdims=True)
    h2 = _sp(gb_ref[0:1, :] * (z - m) / jnp.sqrt(v + EPS) + gb_ref[1:2, :])
    h2_ref[...] = h2
    ha_ref[...] = jnp.dot(h2, wa_ref[...], preferred_element_type=F32)
    hc_ref[...] = jnp.dot(h2, wc_ref[...], preferred_element_type=F32)


def _p4(h, aggp, w, gb, wa, wc):
    return pl.pallas_call(
        _p4_body,
        out_shape=[jax.ShapeDtypeStruct((N, H), F32)] * 3,
    )(h, aggp, w, gb, wa, wc)


def _final_body(h_ref, ag_ref, w3_ref, gb3_ref, batch_ref, comp_ref,
                cmpw_ref, cmpgb_ref, fc1w_ref, fc1gb_ref, fc2w_ref, fc2gb_ref,
                outw_ref, outb_ref, o_ref):
    hin = h_ref[...] + ag_ref[0:N, :] + ag_ref[NP:NP + N, :]
    z = jnp.dot(hin, w3_ref[...], preferred_element_type=F32)
    m = jnp.mean(z, axis=0, keepdims=True)
    v = jnp.mean((z - m) * (z - m), axis=0, keepdims=True)
    h4 = _sp(gb3_ref[0:1, :] * (z - m) / jnp.sqrt(v + EPS) + gb3_ref[1:2, :])

    seg = jax.lax.broadcasted_iota(jnp.int32, (N, G), 1)
    p = (batch_ref[...] == seg).astype(F32)
    sums = lax.dot_general(p, h4, (((0,), (0,)), ((), ())),
                           preferred_element_type=F32,
                           precision=lax.Precision.HIGHEST)
    cnt = jnp.sum(p, axis=0)[:, None]
    gmean = sums / jnp.maximum(cnt, 1.0)

    def bn_sp(y, gb):
        mm = jnp.mean(y, axis=0, keepdims=True)
        vv = jnp.mean((y - mm) * (y - mm), axis=0, keepdims=True)
        return _sp(gb[0:1, :] * (y - mm) / jnp.sqrt(vv + EPS) + gb[1:2, :])

    cf = bn_sp(jnp.dot(comp_ref[...], cmpw_ref[...],
                       preferred_element_type=F32), cmpgb_ref[...])
    g1 = jnp.concatenate([gmean, cf], axis=1)
    g2 = bn_sp(jnp.dot(g1, fc1w_ref[...], preferred_element_type=F32),
               fc1gb_ref[...])
    g3 = bn_sp(jnp.dot(g2, fc2w_ref[...], preferred_element_type=F32),
               fc2gb_ref[...])
    o_ref[...] = (jnp.dot(g3, outw_ref[...], preferred_element_type=F32)
                  + outb_ref[0:1, :])


def _final(h, aggp, w3, gb3, batch2d, comp, cmpw, cmpgb, fc1w, fc1gb,
           fc2w, fc2gb, outw, outb):
    return pl.pallas_call(
        _final_body,
        out_shape=jax.ShapeDtypeStruct((G, 1), F32),
    )(h, aggp, w3, gb3, batch2d, comp, cmpw, cmpgb, fc1w, fc1gb,
      fc2w, fc2gb, outw, outb)


@functools.cache
def _mesh():
    return plsc.VectorSubcoreMesh(core_axis_name="c", subcore_axis_name="s",
                                  num_cores=NC, num_subcores=NS)


_NCH = EPW // GB


def _gather_body(ha_hbm, hc_hbm, eb_hbm, row_hbm, col_hbm,
                 y1_hbm, stats_hbm,
                 idxr, idxc, bufa, bufc, bufe, bufy, sbuf, sems):
    cid = lax.axis_index("c")
    sid = lax.axis_index("s")
    wid = sid * NC + cid
    base0 = wid * EPW

    pltpu.sync_copy(row_hbm.at[wid], idxr)
    pltpu.sync_copy(col_hbm.at[wid], idxc)

    def issue(c, b):
        base = base0 + c * GB
        isl = pl.ds(c * GB, GB)
        pltpu.async_copy(ha_hbm.at[idxr.at[isl]], bufa.at[b], sems.at[b, 0])
        pltpu.async_copy(hc_hbm.at[idxc.at[isl]], bufc.at[b], sems.at[b, 1])
        pltpu.async_copy(eb_hbm.at[pl.ds(base, GB)], bufe.at[b], sems.at[b, 2])

    def wait_in(b):
        pltpu.make_async_copy(eb_hbm.at[pl.ds(0, GB)], bufa.at[b],
                              sems.at[b, 0]).wait()
        pltpu.make_async_copy(eb_hbm.at[pl.ds(0, GB)], bufc.at[b],
                              sems.at[b, 1]).wait()
        pltpu.make_async_copy(eb_hbm.at[pl.ds(0, GB)], bufe.at[b],
                              sems.at[b, 2]).wait()

    def wait_wb(b):
        pltpu.make_async_copy(eb_hbm.at[pl.ds(0, GB)], bufy.at[b],
                              sems.at[b, 3]).wait()

    def compute(c, b, accs):
        def rowloop(r, acc):
            acc_s, acc_q = acc
            ns, nq = [], []
            for k in range(H // 16):
                sl = pl.ds(16 * k, 16)
                y = bufa[b, r, sl] + bufc[b, r, sl] + bufe[b, r, sl]
                bufy[b, r, sl] = y
                ns.append(acc_s[k] + y)
                nq.append(acc_q[k] + y * y)
            return (tuple(ns), tuple(nq))

        accs = lax.fori_loop(0, GB, rowloop, accs)
        pltpu.async_copy(bufy.at[b], y1_hbm.at[pl.ds(base0 + c * GB, GB)],
                         sems.at[b, 3])
        return accs

    z = jnp.zeros((16,), F32)
    accs = (tuple(z for _ in range(H // 16)), tuple(z for _ in range(H // 16)))

    issue(0, 0)
    issue(1, 1)

    def outer(k2, accs):
        for j in range(2):
            c = 2 * k2 + j
            wait_in(j)

            @pl.when(k2 > 0)
            def _():
                wait_wb(j)

            accs = compute(c, j, accs)

            @pl.when(c + 2 < _NCH)
            def _():
                issue(c + 2, j)
        return accs

    accs = lax.fori_loop(0, (_NCH - 1) // 2, outer, accs)
    wait_in(0)
    wait_wb(0)
    accs = compute(_NCH - 1, 0, accs)
    wait_wb(1)
    wait_wb(0)

    acc_s, acc_q = accs
    for k in range(H // 16):
        sl = pl.ds(16 * k, 16)
        sbuf[0, sl] = acc_s[k]
        sbuf[1, sl] = acc_q[k]
    pltpu.sync_copy(sbuf, stats_hbm.at[pl.ds(wid * 8, 8)])


@functools.cache
def _gather_call():
    return pl.kernel(
        _gather_body,
        out_type=[jax.ShapeDtypeStruct((E, H), F32),
                  jax.ShapeDtypeStruct((NW * 8, H), F32)],
        mesh=_mesh(),
        scratch_types=[
            pltpu.VMEM((EPW,), jnp.int32),
            pltpu.VMEM((EPW,), jnp.int32),
            pltpu.VMEM((2, GB, H), F32),
            pltpu.VMEM((2, GB, H), F32),
            pltpu.VMEM((2, GB, H), F32),
            pltpu.VMEM((2, GB, H), F32),
            pltpu.VMEM((8, H), F32),
            pltpu.SemaphoreType.DMA((2, 4)),
        ],
    )


_NCHS = EPW // SB


def _scatter_body(m_hbm, col_hbm, z_hbm, aggp_hbm, shared, idxall, mbuf, sems):
    cid = lax.axis_index("c")
    sid = lax.axis_index("s")
    wid = sid * NC + cid
    base0 = wid * EPW
    pltpu.sync_copy(col_hbm.at[wid], idxall)
    pltpu.sync_copy(z_hbm.at[pl.ds(sid * RPT, RPT)],
                    shared.at[pl.ds(sid * RPT, RPT)])
    plsc.subcore_barrier()

    def issue(c, b):
        pltpu.async_copy(m_hbm.at[pl.ds(base0 + c * SB, SB)], mbuf.at[b],
                         sems.at[b])

    def wait_in(b):
        pltpu.make_async_copy(m_hbm.at[pl.ds(0, SB)], mbuf.at[b],
                              sems.at[b]).wait()

    issue(0, 0)
    issue(1, 1)

    def outer(k2, carry):
        for j in range(2):
            c = 2 * k2 + j
            wait_in(j)
            pltpu.sync_copy(mbuf.at[j], shared.at[idxall.at[c]], add=True)

            @pl.when(c + 2 < _NCHS)
            def _():
                issue(c + 2, j)
        return carry

    lax.fori_loop(0, (_NCHS - 1) // 2, outer, 0)
    wait_in(0)
    pltpu.sync_copy(mbuf.at[0], shared.at[idxall.at[_NCHS - 1]], add=True)
    plsc.subcore_barrier()
    pltpu.sync_copy(shared.at[pl.ds(sid * RPT, RPT)],
                    aggp_hbm.at[pl.ds(cid * NP + sid * RPT, RPT)])


@functools.cache
def _scatter_call():
    return pl.kernel(
        _scatter_body,
        out_type=jax.ShapeDtypeStruct((NC * NP, H), F32),
        mesh=_mesh(),
        scratch_types=[
            pltpu.VMEM_SHARED((NP, H), F32),
            pltpu.VMEM((_NCHS, SB), jnp.int32),
            pltpu.VMEM((2, SB, H), F32),
            pltpu.SemaphoreType.DMA((2,)),
        ],
    )


def kernel(x, edge_attr, comp_features, params, edge_index, batch):
    p = params
    x = x.astype(F32)
    edge_attr = edge_attr.astype(F32)
    comp_features = comp_features.astype(F32)
    row = edge_index[0].astype(jnp.int32)
    col = edge_index[1].astype(jnp.int32)
    row2 = row.reshape(NW, EPW)
    col2 = col.reshape(NW, EPW)
    col3 = col.reshape(NW, _NCHS, SB)
    batch2d = batch.astype(jnp.int32).reshape(N, 1)
    zeros_n = jnp.zeros((NP, H), F32)

    convs = p['convs']
    was = [c['e1_W'][0:H].astype(F32) for c in convs]
    wbs = [c['e1_W'][H:2 * H].astype(F32) for c in convs]
    wcs = [c['e1_W'][2 * H:3 * H].astype(F32) for c in convs]

    def gb(gname, bname, src):
        return jnp.stack([src[gname], src[bname]]).reshape(2, H).astype(F32)

    ne_gb = jnp.stack([p['ne_g'], p['ne_beta']]).reshape(2, H).astype(F32)
    h, ha, hc = _node_enc(x, p['ne_W'].astype(F32), ne_gb, was[0], wcs[0])

    w_ee = p['ee_W'].astype(F32)
    est = _edge_stats(edge_attr, w_ee)
    mean_y = est[0] / E
    var_y = est[1] / E - mean_y * mean_y
    scale = p['ee_g'] / jnp.sqrt(var_y + EPS)
    shift = p['ee_beta'] - mean_y * scale
    ee_ss = jnp.stack([scale, shift]).reshape(2, H).astype(F32)
    ebs = _edge_apply(edge_attr, w_ee, ee_ss, wbs)

    for l in range(4):
        c = convs[l]
        y1, stats = _gather_call()(ha, hc, ebs[l], row2, col2)
        st = stats.reshape(NW, 8, H)
        ss1 = _scale_shift(jnp.sum(st[:, 0], axis=0), jnp.sum(st[:, 1], axis=0),
                           float(E), c['bn1_g'], c['bn1_b'])
        w2 = c['e2_W'].astype(F32)
        st2 = _p2s(y1, ss1, w2)
        ss2 = _scale_shift(st2[0], st2[1], float(E), c['bn2_g'], c['bn2_b'])
        msg = _p2a(y1, ss1, ss2, w2)
        aggp = _scatter_call()(msg, col3, zeros_n)
        gb3 = gb('bn3_g', 'bn3_b', c)
        if l < 3:
            h, ha, hc = _p4(h, aggp, c['n_W'].astype(F32), gb3,
                            was[l + 1], wcs[l + 1])
        else:
            cmp_gb = gb('cmp_g', 'cmp_beta', p)
            fc1_gb = gb('fc1_g', 'fc1_beta', p)
            fc2_gb = gb('fc2_g', 'fc2_beta', p)
            out = _final(h, aggp, c['n_W'].astype(F32), gb3, batch2d,
                         comp_features, p['cmp_W'].astype(F32), cmp_gb,
                         p['fc1_W'].astype(F32), fc1_gb,
                         p['fc2_W'].astype(F32), fc2_gb,
                         p['out_W'].astype(F32),
                         p['out_b'].reshape(1, 1).astype(F32))
    return out
